# baseline (speedup 1.0000x reference)
import functools

import jax
import jax.numpy as jnp
from jax.experimental import pallas as pl
from jax.experimental.pallas import tpu as pltpu

N = 16384
C = 1000
K = int(0.7 * N)
IGNORE = 0

R = 512
NB = N // R


def _stage1_body(x_ref, tgt_ref, loss_ref):
    x = x_ref[...]
    tgt = tgt_ref[...]
    m = jnp.max(x, axis=1, keepdims=True)
    s = jnp.sum(jnp.exp(x - m), axis=1, keepdims=True)
    lse = m + jnp.log(s)
    cols = jax.lax.broadcasted_iota(jnp.int32, (R, C), 1)
    picked = jnp.sum(jnp.where(cols == tgt, x, 0.0), axis=1, keepdims=True)
    loss = lse - picked
    loss_ref[...] = jnp.where(tgt == IGNORE, 0.0, loss)


def _stage2_body(loss_ref, out_ref):
    x = loss_ref[...]
    bits = jax.lax.bitcast_convert_type(x, jnp.int32)

    def body(_, carry):
        lo, hi = carry
        mid = lo + (hi - lo) // 2
        cnt = jnp.sum(jnp.where(bits >= mid, 1, 0))
        return (jnp.where(cnt >= K, mid, lo), jnp.where(cnt >= K, hi, mid))

    lo, _ = jax.lax.fori_loop(
        0, 31, body, (jnp.int32(0), jnp.int32(0x7F800001)))
    t = lo
    gt = bits > t
    cnt_gt = jnp.sum(jnp.where(gt, 1.0, 0.0))
    sum_gt = jnp.sum(jnp.where(gt, x, 0.0))
    tmat = jax.lax.bitcast_convert_type(
        jnp.full((8, 128), t, jnp.int32), jnp.float32)
    tv = jnp.max(tmat)
    out_ref[0, 0] = (sum_gt + (jnp.float32(K) - cnt_gt) * tv) * (1.0 / K)


@jax.jit
def kernel(input, target):
    tgt2d = target.astype(jnp.int32)[:, None]

    loss2d = pl.pallas_call(
        _stage1_body,
        grid=(NB,),
        in_specs=[
            pl.BlockSpec((R, C), lambda i: (i, 0)),
            pl.BlockSpec((R, 1), lambda i: (i, 0)),
        ],
        out_specs=pl.BlockSpec((R, 1), lambda i: (i, 0)),
        out_shape=jax.ShapeDtypeStruct((N, 1), jnp.float32),
    )(input, tgt2d)

    loss_sq = loss2d.reshape(128, 128)
    out = pl.pallas_call(
        _stage2_body,
        out_specs=pl.BlockSpec(memory_space=pltpu.SMEM),
        out_shape=jax.ShapeDtypeStruct((1, 1), jnp.float32),
    )(loss_sq)
    return out[0, 0]


# speedup vs baseline: 3.6266x; 3.6266x over previous
import jax
import jax.numpy as jnp
from jax.experimental import pallas as pl
from jax.experimental.pallas import tpu as pltpu

N = 16384
C = 1000
K = int(0.7 * N)
IGNORE = 0

B = 2048
NB = N // B


def _body(xt_ref, tgt_ref, out_ref, loss_ref):
    i = pl.program_id(0)
    x = xt_ref[...]
    tgt = tgt_ref[...]
    m = jnp.max(x, axis=0, keepdims=True)
    s = jnp.sum(jnp.exp(x - m), axis=0, keepdims=True)
    lse = m + jnp.log(s)
    rows = jax.lax.broadcasted_iota(jnp.int32, (C, B), 0)
    picked = jnp.sum(jnp.where(rows == tgt, x, 0.0), axis=0, keepdims=True)
    loss_ref[pl.ds(i, 1), :] = jnp.where(tgt == IGNORE, 0.0, lse - picked)

    @pl.when(i == NB - 1)
    def _select():
        lx = loss_ref[...]
        bits = jax.lax.bitcast_convert_type(lx, jnp.int32)

        def srch(_, carry):
            lo, hi = carry
            mid = lo + (hi - lo) // 2
            cnt = jnp.sum(jnp.where(bits >= mid, 1, 0))
            return (jnp.where(cnt >= K, mid, lo),
                    jnp.where(cnt >= K, hi, mid))

        t, _ = jax.lax.fori_loop(
            0, 31, srch, (jnp.int32(0), jnp.int32(0x7F800001)))
        gt = bits > t
        cnt_gt = jnp.sum(jnp.where(gt, 1.0, 0.0))
        sum_gt = jnp.sum(jnp.where(gt, lx, 0.0))
        tv = jnp.max(jax.lax.bitcast_convert_type(
            jnp.full((8, 128), t, jnp.int32), jnp.float32))
        out_ref[0, 0] = (sum_gt + (jnp.float32(K) - cnt_gt) * tv) * (1.0 / K)


@jax.jit
def kernel(input, target):
    xt = input.T
    tgt2d = target.astype(jnp.int32)[None, :]

    out = pl.pallas_call(
        _body,
        grid=(NB,),
        in_specs=[
            pl.BlockSpec((C, B), lambda i: (0, i)),
            pl.BlockSpec((1, B), lambda i: (0, i)),
        ],
        out_specs=pl.BlockSpec(memory_space=pltpu.SMEM),
        out_shape=jax.ShapeDtypeStruct((1, 1), jnp.float32),
        scratch_shapes=[pltpu.VMEM((NB, B), jnp.float32)],
    )(xt, tgt2d)
    return out[0, 0]
